# baseline (speedup 1.0000x reference)
import functools

import jax
import jax.numpy as jnp
from jax import lax
from jax.experimental import pallas as pl
from jax.experimental.pallas import tpu as pltpu
from jax.experimental.pallas import tpu_sc as plsc

B = 1024
L = 50
N = 100000
D = 128
S = 12
FLAT = B * L
NWORK = 32
PER_W = FLAT // NWORK
CH = 400
NCH = PER_W // CH
EV1_PER_W = S * FLAT // NWORK
EV1_NCH = EV1_PER_W // CH


def _sc_gather_a(inputs_flat, item_flat, side_i, embedding):
    info = plsc.get_sparse_core_info()
    nc = info.num_cores
    mesh = plsc.VectorSubcoreMesh(core_axis_name="c", subcore_axis_name="s")

    @functools.partial(
        pl.kernel,
        mesh=mesh,
        compiler_params=pltpu.CompilerParams(needs_layout_passes=False),
        out_type=[
            jax.ShapeDtypeStruct((FLAT, 128), jnp.int32),
            jax.ShapeDtypeStruct((FLAT, D), jnp.float32),
            jax.ShapeDtypeStruct((FLAT, D), jnp.float32),
        ],
        scratch_types=[
            pltpu.VMEM((CH,), jnp.int32),
            pltpu.VMEM((CH, 128), jnp.int32),
            pltpu.VMEM((CH, D), jnp.float32),
            pltpu.SemaphoreType.DMA,
        ],
    )
    def ka(inp_h, item_h, side_h, emb_h, sideg_o, ev0_o, item_o,
           ivc, cbuf, ebuf, sem):
        wid = lax.axis_index("s") * nc + lax.axis_index("c")
        base = wid * PER_W

        for c in range(NCH):
            off = base + c * CH
            pltpu.sync_copy(inp_h.at[pl.ds(off, CH)], ivc)
            pltpu.async_copy(side_h.at[ivc], cbuf, sem).wait()
            pltpu.sync_copy(cbuf, sideg_o.at[pl.ds(off, CH)])
            pltpu.async_copy(emb_h.at[ivc], ebuf, sem).wait()
            pltpu.sync_copy(ebuf, ev0_o.at[pl.ds(off, CH)])
        for c in range(NCH):
            off = base + c * CH
            pltpu.sync_copy(item_h.at[pl.ds(off, CH)], ivc)
            pltpu.async_copy(emb_h.at[ivc], ebuf, sem).wait()
            pltpu.sync_copy(ebuf, item_o.at[pl.ds(off, CH)])

    return ka(inputs_flat, item_flat, side_i, embedding)


NSPLIT = 2
HALF = FLAT // NSPLIT
CPP = HALF // CH
NCHK = S * CPP // NWORK


def _sc_gather_b(nbidx, embedding, h):
    info = plsc.get_sparse_core_info()
    nc = info.num_cores
    mesh = plsc.VectorSubcoreMesh(core_axis_name="c", subcore_axis_name="s")
    lo = h * HALF

    @functools.partial(
        pl.kernel,
        mesh=mesh,
        compiler_params=pltpu.CompilerParams(needs_layout_passes=False),
        out_type=jax.ShapeDtypeStruct((S, HALF, D), jnp.float32),
        scratch_types=[
            pltpu.VMEM((CH,), jnp.int32),
            pltpu.VMEM((CH,), jnp.int32),
            pltpu.VMEM((CH, D), jnp.float32),
            pltpu.VMEM((CH, D), jnp.float32),
            pltpu.SemaphoreType.DMA,
            pltpu.SemaphoreType.DMA,
        ],
    )
    def kb(nbidx_h, emb_h, ev1_o, cidx0, cidx1, ebuf0, ebuf1, sem0, sem1):
        wid = lax.axis_index("s") * nc + lax.axis_index("c")
        kbase = wid * NCHK
        cids = (cidx0, cidx1)
        ebufs = (ebuf0, ebuf1)
        sems = (sem0, sem1)

        def in_off(k):
            return (k // CPP) * FLAT + lo + (k % CPP) * CH

        pltpu.sync_copy(nbidx_h.at[pl.ds(in_off(kbase), CH)], cidx0)
        pltpu.async_copy(emb_h.at[cidx0], ebuf0, sem0)

        def body(j, _):
            k = kbase + j

            def step(cur, nxt):
                @pl.when(j < NCHK - 1)
                def _():
                    pltpu.sync_copy(
                        nbidx_h.at[pl.ds(in_off(k + 1), CH)], cids[nxt])
                    pltpu.async_copy(emb_h.at[cids[nxt]], ebufs[nxt], sems[nxt])
                pltpu.make_async_copy(
                    emb_h.at[cids[cur]], ebufs[cur], sems[cur]).wait()
                pltpu.sync_copy(
                    ebufs[cur],
                    ev1_o.at[k // CPP, pl.ds((k % CPP) * CH, CH)])
            lax.cond(j % 2 == 0, lambda: step(0, 1), lambda: step(1, 0))
            return 0

        lax.fori_loop(0, NCHK, body, 0)

    return kb(nbidx, embedding)


GB = 16
R = GB * L


def _tc_body(ev1_r, ev0_r, item_r, nw_r, mask_r, rept_r, rep_r,
             w1d_r, w1l_r, w2_r, w3a_r, w3b_r, out_r):
    mask_blk = mask_r[...]
    m_tiled = jnp.concatenate([mask_blk] * GB, axis=1)
    m_mat = rept_r[...] * m_tiled
    sess = jnp.dot(m_mat, item_r[...], preferred_element_type=jnp.float32)
    denom = jnp.sum(mask_blk, axis=1, keepdims=True)
    sess = sess / denom
    sess_rep = jnp.dot(rep_r[...], sess, preferred_element_type=jnp.float32)


    ev3 = ev1_r[...]
    prods = (ev3 * sess_rep[None]).reshape(S * R, D)
    pre = jnp.dot(prods, w1d_r[...], preferred_element_type=jnp.float32)
    nw = nw_r[...]
    w1l = w1l_r[...]
    segs = []
    for s in range(S):
        seg = pre[s * R:(s + 1) * R] + nw[:, s:s + 1] * w1l
        segs.append(jnp.where(seg >= 0, seg, 0.2 * seg))
    pre2 = jnp.concatenate(segs, axis=0)
    sc = jnp.dot(pre2, w2_r[...], preferred_element_type=jnp.float32)
    scs = [sc[s * R:(s + 1) * R] for s in range(S)]
    m = scs[0]
    for s in range(1, S):
        m = jnp.maximum(m, scs[s])
    es = [jnp.exp(x - m) for x in scs]
    den = es[0]
    for s in range(1, S):
        den = den + es[s]
    inv = 1.0 / den
    nv = (es[0] * inv) * ev1_r[0]
    for s in range(1, S):
        nv = nv + (es[s] * inv) * ev1_r[s]
    out = (jnp.dot(ev0_r[...], w3a_r[...], preferred_element_type=jnp.float32)
           + jnp.dot(nv, w3b_r[...], preferred_element_type=jnp.float32))
    out_r[...] = jnp.maximum(out, 0.0)


def _tc_specs(h):
    nblk = HALF // R
    grid = (nblk,)
    in_specs = [
        pl.BlockSpec((S, R, D), lambda c: (0, c, 0)),
        pl.BlockSpec((R, D), lambda c: (c + h * nblk, 0)),
        pl.BlockSpec((R, D), lambda c: (c + h * nblk, 0)),
        pl.BlockSpec((R, S), lambda c: (c + h * nblk, 0)),
        pl.BlockSpec((GB, L), lambda c: (c + h * nblk, 0)),
        pl.BlockSpec((GB, R), lambda c: (0, 0)),
        pl.BlockSpec((R, GB), lambda c: (0, 0)),
        pl.BlockSpec((D, D), lambda c: (0, 0)),
        pl.BlockSpec((1, D), lambda c: (0, 0)),
        pl.BlockSpec((D, 1), lambda c: (0, 0)),
        pl.BlockSpec((D, D), lambda c: (0, 0)),
        pl.BlockSpec((D, D), lambda c: (0, 0)),
    ]
    out_specs = pl.BlockSpec((R, D), lambda c: (c, 0))
    return grid, in_specs, out_specs


def kernel(inputs, mask_item, item, embedding, adj_all, num_w, w_1, w_2, w_3):
    inputs_flat = inputs.reshape(-1).astype(jnp.int32)
    item_flat = item.reshape(-1).astype(jnp.int32)
    side_i = jnp.concatenate(
        [adj_all.astype(jnp.int32),
         lax.bitcast_convert_type(num_w, jnp.int32),
         jnp.zeros((N, 128 - 2 * S), jnp.int32)], axis=1)

    sideg, ev0, item_emb = _sc_gather_a(inputs_flat, item_flat, side_i, embedding)

    nbidx = sideg[:, :S].T.reshape(-1)
    neigh_w = lax.bitcast_convert_type(sideg[:, S:2 * S], jnp.float32)

    ev1s = [_sc_gather_b(nbidx, embedding, h) for h in range(NSPLIT)]

    rows = jnp.arange(R, dtype=jnp.int32)[None, :] // L
    rept = (rows == jnp.arange(GB, dtype=jnp.int32)[:, None]).astype(jnp.float32)
    rep = rept.T

    w1d, w1l = w_1[:D], w_1[D:D + 1]
    w3a, w3b = w_3[:D], w_3[D:]

    outs = []
    for h in range(NSPLIT):
        grid, in_specs, out_specs = _tc_specs(h)
        outs.append(pl.pallas_call(
            _tc_body,
            grid=grid,
            in_specs=in_specs,
            out_specs=out_specs,
            out_shape=jax.ShapeDtypeStruct((HALF, D), jnp.float32),
        )(ev1s[h], ev0, item_emb, neigh_w, mask_item, rept, rep,
          w1d, w1l, w_2, w3a, w3b))
    return jnp.concatenate(outs, axis=0).reshape(B, L, D)


# speedup vs baseline: 1.0002x; 1.0002x over previous
import functools

import jax
import jax.numpy as jnp
from jax import lax
from jax.experimental import pallas as pl
from jax.experimental.pallas import tpu as pltpu
from jax.experimental.pallas import tpu_sc as plsc

B = 1024
L = 50
N = 100000
D = 128
S = 12
FLAT = B * L
NWORK = 32
PER_W = FLAT // NWORK
CH = 400
NCH = PER_W // CH
EV1_PER_W = S * FLAT // NWORK
EV1_NCH = EV1_PER_W // CH


def _sc_gather_a(inputs_flat, item_flat, side_i, embedding):
    info = plsc.get_sparse_core_info()
    nc = info.num_cores
    mesh = plsc.VectorSubcoreMesh(core_axis_name="c", subcore_axis_name="s")

    @functools.partial(
        pl.kernel,
        mesh=mesh,
        compiler_params=pltpu.CompilerParams(needs_layout_passes=False),
        out_type=[
            jax.ShapeDtypeStruct((FLAT, 128), jnp.int32),
            jax.ShapeDtypeStruct((FLAT, D), jnp.float32),
            jax.ShapeDtypeStruct((FLAT, D), jnp.float32),
        ],
        scratch_types=[
            pltpu.VMEM((CH,), jnp.int32),
            pltpu.VMEM((CH, 128), jnp.int32),
            pltpu.VMEM((CH, D), jnp.float32),
            pltpu.SemaphoreType.DMA,
        ],
    )
    def ka(inp_h, item_h, side_h, emb_h, sideg_o, ev0_o, item_o,
           ivc, cbuf, ebuf, sem):
        wid = lax.axis_index("s") * nc + lax.axis_index("c")
        base = wid * PER_W

        for c in range(NCH):
            off = base + c * CH
            pltpu.sync_copy(inp_h.at[pl.ds(off, CH)], ivc)
            pltpu.async_copy(side_h.at[ivc], cbuf, sem).wait()
            pltpu.sync_copy(cbuf, sideg_o.at[pl.ds(off, CH)])
            pltpu.async_copy(emb_h.at[ivc], ebuf, sem).wait()
            pltpu.sync_copy(ebuf, ev0_o.at[pl.ds(off, CH)])
        for c in range(NCH):
            off = base + c * CH
            pltpu.sync_copy(item_h.at[pl.ds(off, CH)], ivc)
            pltpu.async_copy(emb_h.at[ivc], ebuf, sem).wait()
            pltpu.sync_copy(ebuf, item_o.at[pl.ds(off, CH)])

    return ka(inputs_flat, item_flat, side_i, embedding)


NSPLIT = 2
HALF = FLAT // NSPLIT
CPP = HALF // CH
NCHK = S * CPP // NWORK


def _sc_gather_b(nbidx, embedding, h):
    info = plsc.get_sparse_core_info()
    nc = info.num_cores
    mesh = plsc.VectorSubcoreMesh(core_axis_name="c", subcore_axis_name="s")
    lo = h * HALF

    @functools.partial(
        pl.kernel,
        mesh=mesh,
        compiler_params=pltpu.CompilerParams(needs_layout_passes=False),
        out_type=jax.ShapeDtypeStruct((S, HALF, D), jnp.float32),
        scratch_types=[
            pltpu.VMEM((CH,), jnp.int32),
            pltpu.VMEM((CH,), jnp.int32),
            pltpu.VMEM((CH, D), jnp.float32),
            pltpu.VMEM((CH, D), jnp.float32),
            pltpu.SemaphoreType.DMA,
            pltpu.SemaphoreType.DMA,
        ],
    )
    def kb(nbidx_h, emb_h, ev1_o, cidx0, cidx1, ebuf0, ebuf1, sem0, sem1):
        wid = lax.axis_index("s") * nc + lax.axis_index("c")
        kbase = wid * NCHK
        cids = (cidx0, cidx1)
        ebufs = (ebuf0, ebuf1)
        sems = (sem0, sem1)

        def in_off(k):
            return (k // CPP) * FLAT + lo + (k % CPP) * CH

        pltpu.sync_copy(nbidx_h.at[pl.ds(in_off(kbase), CH)], cidx0)
        pltpu.async_copy(emb_h.at[cidx0], ebuf0, sem0)

        def body(j, _):
            k = kbase + j

            def step(cur, nxt):
                @pl.when(j < NCHK - 1)
                def _():
                    pltpu.sync_copy(
                        nbidx_h.at[pl.ds(in_off(k + 1), CH)], cids[nxt])
                    pltpu.async_copy(emb_h.at[cids[nxt]], ebufs[nxt], sems[nxt])
                pltpu.make_async_copy(
                    emb_h.at[cids[cur]], ebufs[cur], sems[cur]).wait()
                pltpu.sync_copy(
                    ebufs[cur],
                    ev1_o.at[k // CPP, pl.ds((k % CPP) * CH, CH)])
            lax.cond(j % 2 == 0, lambda: step(0, 1), lambda: step(1, 0))
            return 0

        lax.fori_loop(0, NCHK, body, 0)

    return kb(nbidx, embedding)


GB = 16
R = GB * L


def _tc_body(ev1_r, ev0_r, item_r, nw_r, mask_r, rept_r, rep_r,
             w1d_r, w1l_r, w2_r, w3a_r, w3b_r, out_r):
    mask_blk = mask_r[...]
    m_tiled = jnp.concatenate([mask_blk] * GB, axis=1)
    m_mat = rept_r[...] * m_tiled
    sess = jnp.dot(m_mat, item_r[...], preferred_element_type=jnp.float32)
    denom = jnp.sum(mask_blk, axis=1, keepdims=True)
    sess = sess / denom
    sess_rep = jnp.dot(rep_r[...], sess, preferred_element_type=jnp.float32)


    ev3 = ev1_r[...]
    prods = (ev3 * sess_rep[None]).reshape(S * R, D)
    pre = jnp.dot(prods, w1d_r[...], preferred_element_type=jnp.float32)
    nw = nw_r[...]
    w1l = w1l_r[...]
    segs = []
    for s in range(S):
        seg = pre[s * R:(s + 1) * R] + nw[:, s:s + 1] * w1l
        segs.append(jnp.where(seg >= 0, seg, 0.2 * seg))
    pre2 = jnp.concatenate(segs, axis=0)
    sc = jnp.dot(pre2, w2_r[...], preferred_element_type=jnp.float32)
    scs = [sc[s * R:(s + 1) * R] for s in range(S)]
    m = scs[0]
    for s in range(1, S):
        m = jnp.maximum(m, scs[s])
    es = [jnp.exp(x - m) for x in scs]
    den = es[0]
    for s in range(1, S):
        den = den + es[s]
    inv = 1.0 / den
    nv = (es[0] * inv) * ev1_r[0]
    for s in range(1, S):
        nv = nv + (es[s] * inv) * ev1_r[s]
    out = (jnp.dot(ev0_r[...], w3a_r[...], preferred_element_type=jnp.float32)
           + jnp.dot(nv, w3b_r[...], preferred_element_type=jnp.float32))
    out_r[...] = jnp.maximum(out, 0.0)


def _tc_specs(h):
    nblk = HALF // R
    grid = (nblk,)
    in_specs = [
        pl.BlockSpec((S, R, D), lambda c: (0, c, 0)),
        pl.BlockSpec((R, D), lambda c: (c + h * nblk, 0)),
        pl.BlockSpec((R, D), lambda c: (c + h * nblk, 0)),
        pl.BlockSpec((R, S), lambda c: (c + h * nblk, 0)),
        pl.BlockSpec((GB, L), lambda c: (c + h * nblk, 0)),
        pl.BlockSpec((GB, R), lambda c: (0, 0)),
        pl.BlockSpec((R, GB), lambda c: (0, 0)),
        pl.BlockSpec((D, D), lambda c: (0, 0)),
        pl.BlockSpec((1, D), lambda c: (0, 0)),
        pl.BlockSpec((D, 1), lambda c: (0, 0)),
        pl.BlockSpec((D, D), lambda c: (0, 0)),
        pl.BlockSpec((D, D), lambda c: (0, 0)),
    ]
    out_specs = pl.BlockSpec((R, D), lambda c: (c, 0))
    return grid, in_specs, out_specs


PBLK = 2000


def _prep_side_body(adj_r, nw_r, out_r):
    nwi = lax.bitcast_convert_type(nw_r[...], jnp.int32)
    pad = jnp.zeros((PBLK, 128 - 2 * S), jnp.int32)
    out_r[...] = jnp.concatenate([adj_r[...], nwi, pad], axis=1)


def _prep_side(adj_all, num_w):
    return pl.pallas_call(
        _prep_side_body,
        grid=(N // PBLK,),
        in_specs=[pl.BlockSpec((PBLK, S), lambda c: (c, 0)),
                  pl.BlockSpec((PBLK, S), lambda c: (c, 0))],
        out_specs=pl.BlockSpec((PBLK, 128), lambda c: (c, 0)),
        out_shape=jax.ShapeDtypeStruct((N, 128), jnp.int32),
    )(adj_all, num_w)


PBLK2 = 2048


def _prep_split_body(sideg_r, nbt_r, nw_r):
    blk = sideg_r[...]
    ids = lax.bitcast_convert_type(blk[:, :S], jnp.float32)
    nbt_r[...] = lax.bitcast_convert_type(ids.T, jnp.int32)
    nw_r[...] = lax.bitcast_convert_type(blk[:, S:2 * S], jnp.float32)


def _prep_split(sideg):
    return pl.pallas_call(
        _prep_split_body,
        grid=(FLAT // PBLK2,),
        in_specs=[pl.BlockSpec((PBLK2, 128), lambda c: (c, 0))],
        out_specs=[pl.BlockSpec((S, PBLK2), lambda c: (0, c)),
                   pl.BlockSpec((PBLK2, S), lambda c: (c, 0))],
        out_shape=[jax.ShapeDtypeStruct((S, FLAT), jnp.int32),
                   jax.ShapeDtypeStruct((FLAT, S), jnp.float32)],
    )(sideg)


def kernel(inputs, mask_item, item, embedding, adj_all, num_w, w_1, w_2, w_3):
    inputs_flat = inputs.reshape(-1).astype(jnp.int32)
    item_flat = item.reshape(-1).astype(jnp.int32)
    side_i = _prep_side(adj_all.astype(jnp.int32), num_w)

    sideg, ev0, item_emb = _sc_gather_a(inputs_flat, item_flat, side_i, embedding)

    nbidxT, neigh_w = _prep_split(sideg)
    nbidx = nbidxT.reshape(-1)

    ev1s = [_sc_gather_b(nbidx, embedding, h) for h in range(NSPLIT)]

    rows = jnp.arange(R, dtype=jnp.int32)[None, :] // L
    rept = (rows == jnp.arange(GB, dtype=jnp.int32)[:, None]).astype(jnp.float32)
    rep = rept.T

    w1d, w1l = w_1[:D], w_1[D:D + 1]
    w3a, w3b = w_3[:D], w_3[D:]

    outs = []
    for h in range(NSPLIT):
        grid, in_specs, out_specs = _tc_specs(h)
        outs.append(pl.pallas_call(
            _tc_body,
            grid=grid,
            in_specs=in_specs,
            out_specs=out_specs,
            out_shape=jax.ShapeDtypeStruct((HALF, D), jnp.float32),
        )(ev1s[h], ev0, item_emb, neigh_w, mask_item, rept, rep,
          w1d, w1l, w_2, w3a, w3b))
    return jnp.concatenate(outs, axis=0).reshape(B, L, D)


# speedup vs baseline: 1.0279x; 1.0277x over previous
import functools

import jax
import jax.numpy as jnp
from jax import lax
from jax.experimental import pallas as pl
from jax.experimental.pallas import tpu as pltpu
from jax.experimental.pallas import tpu_sc as plsc

B = 1024
L = 50
N = 100000
D = 128
S = 12
FLAT = B * L
NWORK = 32
PER_W = FLAT // NWORK
CH = 400
NCH = PER_W // CH
EV1_PER_W = S * FLAT // NWORK
EV1_NCH = EV1_PER_W // CH


def _sc_gather_a(inputs_flat, item_flat, side_i, embedding):
    info = plsc.get_sparse_core_info()
    nc = info.num_cores
    mesh = plsc.VectorSubcoreMesh(core_axis_name="c", subcore_axis_name="s")

    @functools.partial(
        pl.kernel,
        mesh=mesh,
        compiler_params=pltpu.CompilerParams(needs_layout_passes=False),
        out_type=[
            jax.ShapeDtypeStruct((FLAT, 128), jnp.int32),
            jax.ShapeDtypeStruct((FLAT, D), jnp.float32),
            jax.ShapeDtypeStruct((FLAT, D), jnp.float32),
        ],
        scratch_types=[
            pltpu.VMEM((CH,), jnp.int32),
            pltpu.VMEM((CH, 128), jnp.int32),
            pltpu.VMEM((CH, D), jnp.float32),
            pltpu.SemaphoreType.DMA,
        ],
    )
    def ka(inp_h, item_h, side_h, emb_h, sideg_o, ev0_o, item_o,
           ivc, cbuf, ebuf, sem):
        wid = lax.axis_index("s") * nc + lax.axis_index("c")
        base = wid * PER_W

        for c in range(NCH):
            off = base + c * CH
            pltpu.sync_copy(inp_h.at[pl.ds(off, CH)], ivc)
            pltpu.async_copy(side_h.at[ivc], cbuf, sem).wait()
            pltpu.sync_copy(cbuf, sideg_o.at[pl.ds(off, CH)])
            pltpu.async_copy(emb_h.at[ivc], ebuf, sem).wait()
            pltpu.sync_copy(ebuf, ev0_o.at[pl.ds(off, CH)])
        for c in range(NCH):
            off = base + c * CH
            pltpu.sync_copy(item_h.at[pl.ds(off, CH)], ivc)
            pltpu.async_copy(emb_h.at[ivc], ebuf, sem).wait()
            pltpu.sync_copy(ebuf, item_o.at[pl.ds(off, CH)])

    return ka(inputs_flat, item_flat, side_i, embedding)


NSPLIT = 4
HALF = FLAT // NSPLIT
CPP = HALF // CH
NCHK = S * CPP // NWORK


def _sc_gather_b(nbidx, embedding, h):
    info = plsc.get_sparse_core_info()
    nc = info.num_cores
    mesh = plsc.VectorSubcoreMesh(core_axis_name="c", subcore_axis_name="s")
    lo = h * HALF

    @functools.partial(
        pl.kernel,
        mesh=mesh,
        compiler_params=pltpu.CompilerParams(needs_layout_passes=False),
        out_type=jax.ShapeDtypeStruct((S, HALF, D), jnp.float32),
        scratch_types=[
            pltpu.VMEM((CH,), jnp.int32),
            pltpu.VMEM((CH,), jnp.int32),
            pltpu.VMEM((CH, D), jnp.float32),
            pltpu.VMEM((CH, D), jnp.float32),
            pltpu.SemaphoreType.DMA,
            pltpu.SemaphoreType.DMA,
        ],
    )
    def kb(nbidx_h, emb_h, ev1_o, cidx0, cidx1, ebuf0, ebuf1, sem0, sem1):
        wid = lax.axis_index("s") * nc + lax.axis_index("c")
        kbase = wid * NCHK
        cids = (cidx0, cidx1)
        ebufs = (ebuf0, ebuf1)
        sems = (sem0, sem1)

        def in_off(k):
            return (k // CPP) * FLAT + lo + (k % CPP) * CH

        pltpu.sync_copy(nbidx_h.at[pl.ds(in_off(kbase), CH)], cidx0)
        pltpu.async_copy(emb_h.at[cidx0], ebuf0, sem0)

        def body(j, _):
            k = kbase + j

            def step(cur, nxt):
                @pl.when(j < NCHK - 1)
                def _():
                    pltpu.sync_copy(
                        nbidx_h.at[pl.ds(in_off(k + 1), CH)], cids[nxt])
                    pltpu.async_copy(emb_h.at[cids[nxt]], ebufs[nxt], sems[nxt])
                pltpu.make_async_copy(
                    emb_h.at[cids[cur]], ebufs[cur], sems[cur]).wait()
                pltpu.sync_copy(
                    ebufs[cur],
                    ev1_o.at[k // CPP, pl.ds((k % CPP) * CH, CH)])
            lax.cond(j % 2 == 0, lambda: step(0, 1), lambda: step(1, 0))
            return 0

        lax.fori_loop(0, NCHK, body, 0)

    return kb(nbidx, embedding)


GB = 16
R = GB * L


def _tc_body(ev1_r, ev0_r, item_r, nw_r, mask_r, rept_r, rep_r,
             w1d_r, w1l_r, w2_r, w3a_r, w3b_r, out_r):
    mask_blk = mask_r[...]
    m_tiled = jnp.concatenate([mask_blk] * GB, axis=1)
    m_mat = rept_r[...] * m_tiled
    sess = jnp.dot(m_mat, item_r[...], preferred_element_type=jnp.float32)
    denom = jnp.sum(mask_blk, axis=1, keepdims=True)
    sess = sess / denom
    sess_rep = jnp.dot(rep_r[...], sess, preferred_element_type=jnp.float32)


    ev3 = ev1_r[...]
    prods = (ev3 * sess_rep[None]).reshape(S * R, D)
    pre = jnp.dot(prods, w1d_r[...], preferred_element_type=jnp.float32)
    nw = nw_r[...]
    w1l = w1l_r[...]
    segs = []
    for s in range(S):
        seg = pre[s * R:(s + 1) * R] + nw[:, s:s + 1] * w1l
        segs.append(jnp.where(seg >= 0, seg, 0.2 * seg))
    pre2 = jnp.concatenate(segs, axis=0)
    sc = jnp.dot(pre2, w2_r[...], preferred_element_type=jnp.float32)
    scs = [sc[s * R:(s + 1) * R] for s in range(S)]
    m = scs[0]
    for s in range(1, S):
        m = jnp.maximum(m, scs[s])
    es = [jnp.exp(x - m) for x in scs]
    den = es[0]
    for s in range(1, S):
        den = den + es[s]
    inv = 1.0 / den
    nv = (es[0] * inv) * ev1_r[0]
    for s in range(1, S):
        nv = nv + (es[s] * inv) * ev1_r[s]
    out = (jnp.dot(ev0_r[...], w3a_r[...], preferred_element_type=jnp.float32)
           + jnp.dot(nv, w3b_r[...], preferred_element_type=jnp.float32))
    out_r[...] = jnp.maximum(out, 0.0)


def _tc_specs(h):
    nblk = HALF // R
    grid = (nblk,)
    in_specs = [
        pl.BlockSpec((S, R, D), lambda c: (0, c, 0)),
        pl.BlockSpec((R, D), lambda c: (c + h * nblk, 0)),
        pl.BlockSpec((R, D), lambda c: (c + h * nblk, 0)),
        pl.BlockSpec((R, S), lambda c: (c + h * nblk, 0)),
        pl.BlockSpec((GB, L), lambda c: (c + h * nblk, 0)),
        pl.BlockSpec((GB, R), lambda c: (0, 0)),
        pl.BlockSpec((R, GB), lambda c: (0, 0)),
        pl.BlockSpec((D, D), lambda c: (0, 0)),
        pl.BlockSpec((1, D), lambda c: (0, 0)),
        pl.BlockSpec((D, 1), lambda c: (0, 0)),
        pl.BlockSpec((D, D), lambda c: (0, 0)),
        pl.BlockSpec((D, D), lambda c: (0, 0)),
    ]
    out_specs = pl.BlockSpec((R, D), lambda c: (c, 0))
    return grid, in_specs, out_specs


PBLK = 2000


def _prep_side_body(adj_r, nw_r, out_r):
    nwi = lax.bitcast_convert_type(nw_r[...], jnp.int32)
    pad = jnp.zeros((PBLK, 128 - 2 * S), jnp.int32)
    out_r[...] = jnp.concatenate([adj_r[...], nwi, pad], axis=1)


def _prep_side(adj_all, num_w):
    return pl.pallas_call(
        _prep_side_body,
        grid=(N // PBLK,),
        in_specs=[pl.BlockSpec((PBLK, S), lambda c: (c, 0)),
                  pl.BlockSpec((PBLK, S), lambda c: (c, 0))],
        out_specs=pl.BlockSpec((PBLK, 128), lambda c: (c, 0)),
        out_shape=jax.ShapeDtypeStruct((N, 128), jnp.int32),
    )(adj_all, num_w)


PBLK2 = 2048


def _prep_split_body(sideg_r, nbt_r, nw_r):
    blk = sideg_r[...]
    ids = lax.bitcast_convert_type(blk[:, :S], jnp.float32)
    nbt_r[...] = lax.bitcast_convert_type(ids.T, jnp.int32)
    nw_r[...] = lax.bitcast_convert_type(blk[:, S:2 * S], jnp.float32)


def _prep_split(sideg):
    return pl.pallas_call(
        _prep_split_body,
        grid=(FLAT // PBLK2,),
        in_specs=[pl.BlockSpec((PBLK2, 128), lambda c: (c, 0))],
        out_specs=[pl.BlockSpec((S, PBLK2), lambda c: (0, c)),
                   pl.BlockSpec((PBLK2, S), lambda c: (c, 0))],
        out_shape=[jax.ShapeDtypeStruct((S, FLAT), jnp.int32),
                   jax.ShapeDtypeStruct((FLAT, S), jnp.float32)],
    )(sideg)


def kernel(inputs, mask_item, item, embedding, adj_all, num_w, w_1, w_2, w_3):
    inputs_flat = inputs.reshape(-1).astype(jnp.int32)
    item_flat = item.reshape(-1).astype(jnp.int32)
    side_i = _prep_side(adj_all.astype(jnp.int32), num_w)

    sideg, ev0, item_emb = _sc_gather_a(inputs_flat, item_flat, side_i, embedding)

    nbidxT, neigh_w = _prep_split(sideg)
    nbidx = nbidxT.reshape(-1)

    ev1s = [_sc_gather_b(nbidx, embedding, h) for h in range(NSPLIT)]

    rows = jnp.arange(R, dtype=jnp.int32)[None, :] // L
    rept = (rows == jnp.arange(GB, dtype=jnp.int32)[:, None]).astype(jnp.float32)
    rep = rept.T

    w1d, w1l = w_1[:D], w_1[D:D + 1]
    w3a, w3b = w_3[:D], w_3[D:]

    outs = []
    for h in range(NSPLIT):
        grid, in_specs, out_specs = _tc_specs(h)
        outs.append(pl.pallas_call(
            _tc_body,
            grid=grid,
            in_specs=in_specs,
            out_specs=out_specs,
            out_shape=jax.ShapeDtypeStruct((HALF, D), jnp.float32),
        )(ev1s[h], ev0, item_emb, neigh_w, mask_item, rept, rep,
          w1d, w1l, w_2, w3a, w3b))
    return jnp.concatenate(outs, axis=0).reshape(B, L, D)
